# baseline (speedup 1.0000x reference)
import functools

import jax
import jax.numpy as jnp
from jax import lax
from jax.experimental import pallas as pl
from jax.experimental.pallas import tpu as pltpu
from jax.experimental.pallas import tpu_sc as plsc

D_MODEL = 1024
D_FF = 4096
N_EXP = 8
TOPK = 2
BLK = 1024
NBLK = D_FF // BLK


def _router_body(token_ref, wr_ref, br_ref, dens_ref, u_ref, idx_ref, val_ref):
    x = token_ref[...]
    logits = jnp.dot(x, wr_ref[...], preferred_element_type=jnp.float32)
    logits = logits + br_ref[...]
    m = jnp.max(logits)
    e = jnp.exp(logits - m)
    sm = e / jnp.sum(e)
    col8 = jax.lax.broadcasted_iota(jnp.int32, (1, N_EXP), 1)
    v0 = jnp.max(sm)
    first_max = jnp.min(jnp.where(sm == v0, col8, N_EXP))
    v1 = jnp.max(jnp.where(col8 == first_max, -jnp.inf, sm))
    val_ref[0] = v0
    val_ref[1] = v1
    flat = dens_ref[...]
    n = N_EXP * N_EXP
    r = jax.lax.broadcasted_iota(jnp.int32, (n, n), 0)
    c = jax.lax.broadcasted_iota(jnp.int32, (n, n), 1)
    tri = (r <= c).astype(jnp.float32)
    cum = jnp.dot(flat, tri, preferred_element_type=jnp.float32)
    col64 = jax.lax.broadcasted_iota(jnp.int32, (1, n), 1)
    c_last = jnp.sum(jnp.where(col64 == n - 1, cum, 0.0))
    u = u_ref[0, 0] * c_last
    idx = jnp.sum((cum < u).astype(jnp.int32))
    i0 = idx // N_EXP
    idx_ref[0] = i0
    idx_ref[1] = idx - N_EXP * i0


def _route(tok2, Wr, br2, dflat, u):
    return pl.pallas_call(
        _router_body,
        out_shape=[
            jax.ShapeDtypeStruct((TOPK,), jnp.int32),
            jax.ShapeDtypeStruct((TOPK,), jnp.float32),
        ],
        in_specs=[
            pl.BlockSpec(memory_space=pltpu.VMEM),
            pl.BlockSpec(memory_space=pltpu.VMEM),
            pl.BlockSpec(memory_space=pltpu.VMEM),
            pl.BlockSpec(memory_space=pltpu.VMEM),
            pl.BlockSpec(memory_space=pltpu.SMEM),
        ],
        out_specs=[
            pl.BlockSpec(memory_space=pltpu.SMEM),
            pl.BlockSpec(memory_space=pltpu.SMEM),
        ],
    )(tok2, Wr, br2, dflat, u)


def _shuffle(x, perm):
    return x.at[perm].get(mode="promise_in_bounds")


def _allreduce(x, op):
    iota = lax.iota(jnp.int32, 16)
    for off in (8, 4, 2, 1):
        x = op(x, _shuffle(x, jnp.bitwise_xor(iota, off)))
    return x


def _sc_router_body(tok_hbm, wrt_hbm, br_hbm, dens_hbm, u_hbm,
                    idx_hbm, val_hbm,
                    tok_v, wrt_v, br_v, dens_v, u_v, idx_v, val_v):
    wid = lax.axis_index("s") * 2 + lax.axis_index("c")

    @pl.when(wid == 0)
    def _():
        pltpu.sync_copy(tok_hbm, tok_v)
        pltpu.sync_copy(wrt_hbm, wrt_v)
        pltpu.sync_copy(br_hbm, br_v)
        pltpu.sync_copy(dens_hbm, dens_v)
        pltpu.sync_copy(u_hbm, u_v)

        iota = lax.iota(jnp.int32, 16)
        neg = jnp.full((16,), -1e30, jnp.float32)

        def body(c, accs):
            t = tok_v[pl.ds(c * 16, 16)]
            return tuple(
                accs[e] + t * wrt_v[pl.ds(c * 16 + e * D_MODEL, 16)]
                for e in range(N_EXP))

        accs = lax.fori_loop(
            0, D_MODEL // 16, body,
            tuple(jnp.zeros((16,), jnp.float32) for _ in range(N_EXP)))
        logits = br_v[...]
        for e in range(N_EXP):
            s = _allreduce(accs[e], jnp.add)
            logits = jnp.where(iota == e, logits + s, logits)

        lm = jnp.where(iota < N_EXP, logits, neg)
        m = _allreduce(lm, jnp.maximum)
        ex = jnp.where(iota < N_EXP, jnp.exp(lm - m), jnp.zeros((16,), jnp.float32))
        sm = ex / _allreduce(ex, jnp.add)
        v0 = _allreduce(sm, jnp.maximum)
        first = _allreduce(jnp.where(sm == v0, iota, jnp.full((16,), 16, jnp.int32)),
                           jnp.minimum)
        v1 = _allreduce(jnp.where(iota == first, neg, sm), jnp.maximum)

        vals = []
        for c in range((N_EXP * N_EXP) // 16):
            ch = dens_v[pl.ds(c * 16, 16)]
            for j in range(16):
                vals.append(ch[j])
        total = jnp.float32(0.0)
        for v in vals:
            total = total + v
        u = u_v[...][0] * total
        c_run = jnp.float32(0.0)
        cnt = jnp.int32(0)
        for v in vals:
            c_run = c_run + v
            cnt = cnt + jnp.where(c_run < u, 1, 0)
        i0 = cnt // N_EXP
        i1 = cnt - N_EXP * i0

        idx_v[...] = jnp.where(iota == 0, i0, i1)
        val_v[...] = jnp.where(iota == 0, v0, v1)
        pltpu.sync_copy(idx_v, idx_hbm)
        pltpu.sync_copy(val_v, val_hbm)


def _sc_route(tokv, wrt, brv, densv, uv):
    mesh = plsc.VectorSubcoreMesh(core_axis_name="c", subcore_axis_name="s")
    k = functools.partial(
        pl.kernel,
        mesh=mesh,
        out_type=[
            jax.ShapeDtypeStruct((16,), jnp.int32),
            jax.ShapeDtypeStruct((16,), jnp.float32),
        ],
        scratch_types=[
            pltpu.VMEM((D_MODEL,), jnp.float32),
            pltpu.VMEM((N_EXP * D_MODEL,), jnp.float32),
            pltpu.VMEM((16,), jnp.float32),
            pltpu.VMEM((N_EXP * N_EXP,), jnp.float32),
            pltpu.VMEM((16,), jnp.float32),
            pltpu.VMEM((16,), jnp.int32),
            pltpu.VMEM((16,), jnp.float32),
        ],
    )(_sc_router_body)
    return k(tokv, wrt, brv, densv, uv)


def _ffn_body(idx_ref, val_ref, token_ref, w1a_ref, w1b_ref, b1_ref,
              w2a_ref, w2b_ref, b2_ref, out_ref):
    e = pl.program_id(0)
    j = pl.program_id(1)

    @pl.when((e == 0) & (j == 0))
    def _():
        out_ref[...] = jnp.zeros_like(out_ref)

    x = token_ref[...]
    half = D_MODEL // 2
    h = (jnp.dot(x[:, :half], w1a_ref[0], preferred_element_type=jnp.float32)
         + jnp.dot(x[:, half:], w1b_ref[0], preferred_element_type=jnp.float32))
    h = jnp.maximum(h + b1_ref[0], 0.0)
    hb = BLK // 2
    part = (jnp.dot(h[:, :hb], w2a_ref[0], preferred_element_type=jnp.float32)
            + jnp.dot(h[:, hb:], w2b_ref[0], preferred_element_type=jnp.float32))
    s = val_ref[e]
    out_ref[...] += s * part

    @pl.when(j == 0)
    def _():
        out_ref[...] += s * b2_ref[0]


def _ffn(idx, vals, tok2, W1, b1, W2, b2):
    grid_spec = pltpu.PrefetchScalarGridSpec(
        num_scalar_prefetch=2,
        grid=(TOPK, NBLK),
        in_specs=[
            pl.BlockSpec((1, D_MODEL), lambda e, j, idx, vals: (0, 0)),
            pl.BlockSpec((1, D_MODEL // 2, BLK),
                         lambda e, j, idx, vals: (idx[e], 0, j)),
            pl.BlockSpec((1, D_MODEL // 2, BLK),
                         lambda e, j, idx, vals: (idx[e], 1, j)),
            pl.BlockSpec((1, 1, BLK), lambda e, j, idx, vals: (idx[e], 0, j)),
            pl.BlockSpec((1, BLK // 2, D_MODEL),
                         lambda e, j, idx, vals: (idx[e], 2 * j, 0)),
            pl.BlockSpec((1, BLK // 2, D_MODEL),
                         lambda e, j, idx, vals: (idx[e], 2 * j + 1, 0)),
            pl.BlockSpec((1, 1, D_MODEL), lambda e, j, idx, vals: (idx[e], 0, 0)),
        ],
        out_specs=pl.BlockSpec((1, D_MODEL), lambda e, j, idx, vals: (0, 0)),
    )
    return pl.pallas_call(
        _ffn_body,
        grid_spec=grid_spec,
        out_shape=jax.ShapeDtypeStruct((1, D_MODEL), jnp.float32),
        compiler_params=pltpu.CompilerParams(
            dimension_semantics=("arbitrary", "arbitrary"),
        ),
    )(idx, vals, tok2, W1, W1, b1.reshape(N_EXP, 1, D_FF), W2, W2,
      b2.reshape(N_EXP, 1, D_MODEL))


def kernel(token, Wr, br, W1, b1, W2, b2, density):
    u = jax.random.uniform(jax.random.key(7), dtype=jnp.float32)
    wrt = Wr.T.reshape(N_EXP * D_MODEL)
    brv = jnp.pad(br, (0, 16 - N_EXP))
    densv = density.reshape(N_EXP * N_EXP)
    uv = jnp.full((16,), u, dtype=jnp.float32)
    idx16, val16 = _sc_route(token, wrt, brv, densv, uv)
    idx = idx16[:TOPK]
    vals = val16[:TOPK]
    tok2 = token.reshape(1, D_MODEL)
    out = _ffn(idx, vals, tok2, W1, b1, W2, b2)
    return out.reshape(D_MODEL)


# speedup vs baseline: 1.5584x; 1.5584x over previous
import jax
import jax.numpy as jnp
from jax.experimental import pallas as pl
from jax.experimental.pallas import tpu as pltpu

D_MODEL = 1024
D_FF = 4096
N_EXP = 8
TOPK = 2
BLK = 1024
NBLK = D_FF // BLK
NSTEP = TOPK * NBLK
NBUF = 3


def _routing(token_ref, wr_ref, br_ref, dens_ref, u_ref):
    x = token_ref[...]
    logits = jnp.dot(x, wr_ref[...], preferred_element_type=jnp.float32)
    logits = logits + br_ref[...]
    m = jnp.max(logits)
    e = jnp.exp(logits - m)
    sm = e / jnp.sum(e)
    col8 = jax.lax.broadcasted_iota(jnp.int32, (1, N_EXP), 1)
    v0 = jnp.max(sm)
    first_max = jnp.min(jnp.where(sm == v0, col8, N_EXP))
    v1 = jnp.max(jnp.where(col8 == first_max, -jnp.inf, sm))
    flat = dens_ref[...]
    n = N_EXP * N_EXP
    r = jax.lax.broadcasted_iota(jnp.int32, (n, n), 0)
    c = jax.lax.broadcasted_iota(jnp.int32, (n, n), 1)
    tri = (r <= c).astype(jnp.float32)
    cum = jnp.dot(flat, tri, preferred_element_type=jnp.float32)
    col64 = jax.lax.broadcasted_iota(jnp.int32, (1, n), 1)
    c_last = jnp.sum(jnp.where(col64 == n - 1, cum, 0.0))
    u = u_ref[0, 0] * c_last
    idx = jnp.sum((cum < u).astype(jnp.int32))
    i0 = idx // N_EXP
    i1 = idx - N_EXP * i0
    return i0, i1, v0, v1


def _moe_body(token_ref, wr_ref, br_ref, dens_ref, u_ref,
              w1_any, b1_any, w2_any, b2_any, out_ref,
              w1b, w2b, b1v, b2v, sem1, sem2, semb):
    i0, i1, v0, v1 = _routing(token_ref, wr_ref, br_ref, dens_ref, u_ref)
    eidx = [i0, i1]
    scales = [v0, v1]

    def w1_copy(s):
        e, j = divmod(s, NBLK)
        return pltpu.make_async_copy(
            w1_any.at[eidx[e], :, pl.ds(j * BLK, BLK)],
            w1b.at[s % NBUF], sem1.at[s % NBUF])

    def w2_copy(s):
        e, j = divmod(s, NBLK)
        return pltpu.make_async_copy(
            w2_any.at[eidx[e], pl.ds(j * BLK, BLK), :],
            w2b.at[s % NBUF], sem2.at[s % NBUF])

    bias_copies = [
        pltpu.make_async_copy(b1_any.at[eidx[e]], b1v.at[e], semb)
        for e in range(TOPK)
    ] + [
        pltpu.make_async_copy(b2_any.at[eidx[e]], b2v.at[e], semb)
        for e in range(TOPK)
    ]
    for cp in bias_copies:
        cp.start()
    for s in range(NBUF - 1):
        w1_copy(s).start()
        w2_copy(s).start()
    for cp in bias_copies:
        cp.wait()

    x = token_ref[...]
    acc = v0 * b2v[0] + v1 * b2v[1]
    for s in range(NSTEP):
        if s + NBUF - 1 < NSTEP:
            w1_copy(s + NBUF - 1).start()
            w2_copy(s + NBUF - 1).start()
        w1_copy(s).wait()
        w2_copy(s).wait()
        e, j = divmod(s, NBLK)
        h = jnp.dot(x, w1b[s % NBUF], preferred_element_type=jnp.float32)
        h = jnp.maximum(h + b1v[e, :, pl.ds(j * BLK, BLK)], 0.0)
        acc = acc + scales[e] * jnp.dot(h, w2b[s % NBUF],
                                        preferred_element_type=jnp.float32)
    out_ref[...] = acc


def _moe(tok2, Wr, br2, dflat, u, W1, b1, W2, b2):
    return pl.pallas_call(
        _moe_body,
        out_shape=jax.ShapeDtypeStruct((1, D_MODEL), jnp.float32),
        in_specs=[
            pl.BlockSpec(memory_space=pltpu.VMEM),
            pl.BlockSpec(memory_space=pltpu.VMEM),
            pl.BlockSpec(memory_space=pltpu.VMEM),
            pl.BlockSpec(memory_space=pltpu.VMEM),
            pl.BlockSpec(memory_space=pltpu.SMEM),
            pl.BlockSpec(memory_space=pl.ANY),
            pl.BlockSpec(memory_space=pl.ANY),
            pl.BlockSpec(memory_space=pl.ANY),
            pl.BlockSpec(memory_space=pl.ANY),
        ],
        out_specs=pl.BlockSpec(memory_space=pltpu.VMEM),
        scratch_shapes=[
            pltpu.VMEM((NBUF, D_MODEL, BLK), jnp.float32),
            pltpu.VMEM((NBUF, BLK, D_MODEL), jnp.float32),
            pltpu.VMEM((TOPK, 1, D_FF), jnp.float32),
            pltpu.VMEM((TOPK, 1, D_MODEL), jnp.float32),
            pltpu.SemaphoreType.DMA((NBUF,)),
            pltpu.SemaphoreType.DMA((NBUF,)),
            pltpu.SemaphoreType.DMA,
        ],
    )(tok2, Wr, br2, dflat, u, W1, b1.reshape(N_EXP, 1, D_FF), W2,
      b2.reshape(N_EXP, 1, D_MODEL))


def kernel(token, Wr, br, W1, b1, W2, b2, density):
    u = jax.random.uniform(jax.random.key(7), dtype=jnp.float32)
    u = u.reshape(1, 1)
    tok2 = token.reshape(1, D_MODEL)
    br2 = br.reshape(1, N_EXP)
    dflat = density.reshape(1, N_EXP * N_EXP)
    out = _moe(tok2, Wr, br2, dflat, u, W1, b1, W2, b2)
    return out.reshape(D_MODEL)


# speedup vs baseline: 1.5599x; 1.0009x over previous
import jax
import jax.numpy as jnp
from jax.experimental import pallas as pl
from jax.experimental.pallas import tpu as pltpu

D_MODEL = 1024
D_FF = 4096
N_EXP = 8
TOPK = 2
BLK = 512
NBLK = D_FF // BLK
NSTEP = TOPK * NBLK
NBUF = 6


def _routing(token_ref, wr_ref, br_ref, dens_ref, u_ref):
    x = token_ref[...]
    logits = jnp.dot(x, wr_ref[...], preferred_element_type=jnp.float32)
    logits = logits + br_ref[...]
    m = jnp.max(logits)
    e = jnp.exp(logits - m)
    sm = e / jnp.sum(e)
    col8 = jax.lax.broadcasted_iota(jnp.int32, (1, N_EXP), 1)
    v0 = jnp.max(sm)
    first_max = jnp.min(jnp.where(sm == v0, col8, N_EXP))
    v1 = jnp.max(jnp.where(col8 == first_max, -jnp.inf, sm))
    flat = dens_ref[...]
    n = N_EXP * N_EXP
    r = jax.lax.broadcasted_iota(jnp.int32, (n, n), 0)
    c = jax.lax.broadcasted_iota(jnp.int32, (n, n), 1)
    tri = (r <= c).astype(jnp.float32)
    cum = jnp.dot(flat, tri, preferred_element_type=jnp.float32)
    col64 = jax.lax.broadcasted_iota(jnp.int32, (1, n), 1)
    c_last = jnp.sum(jnp.where(col64 == n - 1, cum, 0.0))
    u = u_ref[0, 0] * c_last
    idx = jnp.sum((cum < u).astype(jnp.int32))
    i0 = idx // N_EXP
    i1 = idx - N_EXP * i0
    return i0, i1, v0, v1


def _moe_body(token_ref, wr_ref, br_ref, dens_ref, u_ref,
              w1_any, b1_any, w2_any, b2_any, out_ref,
              w1b, w2b, b1v, b2v, sem1, sem2, semb):
    i0, i1, v0, v1 = _routing(token_ref, wr_ref, br_ref, dens_ref, u_ref)
    eidx = [i0, i1]
    scales = [v0, v1]

    def w1_copy(s):
        e, j = divmod(s, NBLK)
        return pltpu.make_async_copy(
            w1_any.at[eidx[e], :, pl.ds(j * BLK, BLK)],
            w1b.at[s % NBUF], sem1.at[s % NBUF])

    def w2_copy(s):
        e, j = divmod(s, NBLK)
        return pltpu.make_async_copy(
            w2_any.at[eidx[e], pl.ds(j * BLK, BLK), :],
            w2b.at[s % NBUF], sem2.at[s % NBUF])

    bias_copies = [
        pltpu.make_async_copy(b1_any.at[eidx[e]], b1v.at[e], semb)
        for e in range(TOPK)
    ] + [
        pltpu.make_async_copy(b2_any.at[eidx[e]], b2v.at[e], semb)
        for e in range(TOPK)
    ]
    for cp in bias_copies:
        cp.start()
    for s in range(NBUF - 1):
        w1_copy(s).start()
        w2_copy(s).start()
    for cp in bias_copies:
        cp.wait()

    x = token_ref[...]
    acc = v0 * b2v[0] + v1 * b2v[1]
    for s in range(NSTEP):
        if s + NBUF - 1 < NSTEP:
            w1_copy(s + NBUF - 1).start()
            w2_copy(s + NBUF - 1).start()
        w1_copy(s).wait()
        w2_copy(s).wait()
        e, j = divmod(s, NBLK)
        h = jnp.dot(x, w1b[s % NBUF], preferred_element_type=jnp.float32)
        h = jnp.maximum(h + b1v[e, :, pl.ds(j * BLK, BLK)], 0.0)
        acc = acc + scales[e] * jnp.dot(h, w2b[s % NBUF],
                                        preferred_element_type=jnp.float32)
    out_ref[...] = acc


def _moe(tok2, Wr, br2, dflat, u, W1, b1, W2, b2):
    return pl.pallas_call(
        _moe_body,
        out_shape=jax.ShapeDtypeStruct((1, D_MODEL), jnp.float32),
        in_specs=[
            pl.BlockSpec(memory_space=pltpu.VMEM),
            pl.BlockSpec(memory_space=pltpu.VMEM),
            pl.BlockSpec(memory_space=pltpu.VMEM),
            pl.BlockSpec(memory_space=pltpu.VMEM),
            pl.BlockSpec(memory_space=pltpu.SMEM),
            pl.BlockSpec(memory_space=pl.ANY),
            pl.BlockSpec(memory_space=pl.ANY),
            pl.BlockSpec(memory_space=pl.ANY),
            pl.BlockSpec(memory_space=pl.ANY),
        ],
        out_specs=pl.BlockSpec(memory_space=pltpu.VMEM),
        scratch_shapes=[
            pltpu.VMEM((NBUF, D_MODEL, BLK), jnp.float32),
            pltpu.VMEM((NBUF, BLK, D_MODEL), jnp.float32),
            pltpu.VMEM((TOPK, 1, D_FF), jnp.float32),
            pltpu.VMEM((TOPK, 1, D_MODEL), jnp.float32),
            pltpu.SemaphoreType.DMA((NBUF,)),
            pltpu.SemaphoreType.DMA((NBUF,)),
            pltpu.SemaphoreType.DMA,
        ],
    )(tok2, Wr, br2, dflat, u, W1, b1.reshape(N_EXP, 1, D_FF), W2,
      b2.reshape(N_EXP, 1, D_MODEL))


def kernel(token, Wr, br, W1, b1, W2, b2, density):
    u = jax.random.uniform(jax.random.key(7), dtype=jnp.float32)
    u = u.reshape(1, 1)
    tok2 = token.reshape(1, D_MODEL)
    br2 = br.reshape(1, N_EXP)
    dflat = density.reshape(1, N_EXP * N_EXP)
    out = _moe(tok2, Wr, br2, dflat, u, W1, b1, W2, b2)
    return out.reshape(D_MODEL)
